# baseline (speedup 1.0000x reference)
import functools

import jax
import jax.numpy as jnp
from jax import lax
from jax.experimental import pallas as pl
from jax.experimental.pallas import tpu as pltpu
from jax.experimental.pallas import tpu_sc as plsc

N = 10000
E = 320000
D = 128

NC = 2
NS = 16
NW = NC * NS
CHUNK = 128
CPT = 80
SLAB = 64
EPW = CHUNK * CPT
E_PAD = NW * EPW
ACC_ROWS = 10112
ZPT = ACC_ROWS // NS

_mesh = plsc.VectorSubcoreMesh(core_axis_name="c", subcore_axis_name="s")


@functools.partial(
    pl.kernel,
    out_type=jax.ShapeDtypeStruct((NC, ACC_ROWS, D), jnp.float32),
    mesh=_mesh,
    scratch_types=[
        pltpu.VMEM((SLAB, CHUNK), jnp.int32),
        pltpu.VMEM((SLAB, CHUNK), jnp.int32),
        pltpu.VMEM((CHUNK, D), jnp.float32),
        pltpu.VMEM((CHUNK, D), jnp.float32),
        pltpu.VMEM_SHARED((ACC_ROWS, D), jnp.float32),
        pltpu.SemaphoreType.DMA,
        pltpu.SemaphoreType.DMA,
    ],
)
def _sc_agg(h_hbm, src_hbm, dst_hbm, zeros_hbm, out_hbm,
            src_v, dst_v, buf0, buf1, acc, sem0, sem1):
    cid = lax.axis_index("c")
    sid = lax.axis_index("s")
    wid = sid * NC + cid

    pltpu.sync_copy(zeros_hbm.at[pl.ds(sid * ZPT, ZPT)],
                    acc.at[pl.ds(sid * ZPT, ZPT)])
    plsc.subcore_barrier()

    def process_section(base, nc):
        pltpu.sync_copy(src_hbm.at[wid].at[pl.ds(base, nc)],
                        src_v.at[pl.ds(0, nc)])
        pltpu.sync_copy(dst_hbm.at[wid].at[pl.ds(base, nc)],
                        dst_v.at[pl.ds(0, nc)])
        pltpu.async_copy(h_hbm.at[src_v.at[0]], buf0, sem0)

        @pl.loop(0, nc, step=2)
        def _(j):
            pltpu.async_copy(h_hbm.at[src_v.at[j + 1]], buf1, sem1)
            pltpu.make_async_copy(h_hbm.at[src_v.at[j]], buf0, sem0).wait()
            pltpu.sync_copy(buf0, acc.at[dst_v.at[j]], add=True)

            @pl.when(j + 2 < nc)
            def _():
                pltpu.async_copy(h_hbm.at[src_v.at[j + 2]], buf0, sem0)

            pltpu.make_async_copy(h_hbm.at[src_v.at[j + 1]], buf1, sem1).wait()
            pltpu.sync_copy(buf1, acc.at[dst_v.at[j + 1]], add=True)

    process_section(0, SLAB)
    process_section(SLAB, CPT - SLAB)

    plsc.subcore_barrier()
    pltpu.sync_copy(acc.at[pl.ds(sid * ZPT, ZPT)],
                    out_hbm.at[cid].at[pl.ds(sid * ZPT, ZPT)])


def _tc_body(h_ref, p_ref, w_ref, b_ref, s_ref, o_ref, *, relu):
    z = s_ref[0, 0] * h_ref[...] + (p_ref[0] + p_ref[1])
    y = lax.dot_general(z, w_ref[...], (((1,), (1,)), ((), ())),
                        preferred_element_type=jnp.float32)
    y = y + b_ref[...]
    o_ref[...] = jnp.maximum(y, 0.0) if relu else y


def _tc_layer(relu):
    return pl.pallas_call(
        functools.partial(_tc_body, relu=relu),
        out_shape=jax.ShapeDtypeStruct((N, D), jnp.float32),
    )


def kernel(x, edge_index, W1, b1, eps1, W2, b2, eps2, W3, b3, eps3):
    src = edge_index[0].astype(jnp.int32)
    dst = edge_index[1].astype(jnp.int32)
    pad = E_PAD - E
    src_p = jnp.concatenate([src, jnp.zeros((pad,), jnp.int32)]).reshape(
        NW, CPT, CHUNK)
    pad_dst = N + (jnp.arange(pad, dtype=jnp.int32) % (ACC_ROWS - N))
    dst_p = jnp.concatenate([dst, pad_dst]).reshape(NW, CPT, CHUNK)
    zeros = jnp.zeros((ACC_ROWS, D), jnp.float32)

    h = x
    for W, b, eps, relu in ((W1, b1, eps1, True),
                            (W2, b2, eps2, True),
                            (W3, b3, eps3, False)):
        parts = _sc_agg(h, src_p, dst_p, zeros)[:, :N]
        scale = jnp.reshape(1.0 + eps, (1, 1)).astype(jnp.float32)
        h = _tc_layer(relu)(h, parts, W, b.reshape(1, D), scale)
    return h


# speedup vs baseline: 1.0097x; 1.0097x over previous
import functools

import jax
import jax.numpy as jnp
from jax import lax
from jax.experimental import pallas as pl
from jax.experimental.pallas import tpu as pltpu
from jax.experimental.pallas import tpu_sc as plsc

N = 10000
E = 320000
D = 128

NC = 2
NS = 16
NW = NC * NS
CHUNK = 128
CPT = 80
SLAB = 64
EPW = CHUNK * CPT
E_PAD = NW * EPW
ACC_ROWS = 10112
ZPT = ACC_ROWS // NS

_mesh = plsc.VectorSubcoreMesh(core_axis_name="c", subcore_axis_name="s")


@functools.partial(
    pl.kernel,
    out_type=jax.ShapeDtypeStruct((NC, ACC_ROWS, D), jnp.float32),
    mesh=_mesh,
    scratch_types=[
        pltpu.VMEM((SLAB, CHUNK), jnp.int32),
        pltpu.VMEM((SLAB, CHUNK), jnp.int32),
        pltpu.VMEM((CHUNK, D), jnp.float32),
        pltpu.VMEM((CHUNK, D), jnp.float32),
        pltpu.VMEM_SHARED((ACC_ROWS, D), jnp.float32),
        pltpu.SemaphoreType.DMA,
        pltpu.SemaphoreType.DMA,
        pltpu.SemaphoreType.DMA,
        pltpu.SemaphoreType.DMA,
    ],
)
def _sc_agg(h_hbm, src_hbm, dst_hbm, zeros_hbm, out_hbm,
            src_v, dst_v, buf0, buf1, acc, sem0, sem0b, sem1, sem1b):
    cid = lax.axis_index("c")
    sid = lax.axis_index("s")
    wid = sid * NC + cid

    pltpu.sync_copy(zeros_hbm.at[pl.ds(sid * ZPT, ZPT)],
                    acc.at[pl.ds(sid * ZPT, ZPT)])
    plsc.subcore_barrier()

    def process_section(base, nc):
        pltpu.sync_copy(src_hbm.at[wid].at[pl.ds(base, nc)],
                        src_v.at[pl.ds(0, nc)])
        pltpu.sync_copy(dst_hbm.at[wid].at[pl.ds(base, nc)],
                        dst_v.at[pl.ds(0, nc)])
        H = CHUNK // 2

        def start_gather(j, buf, sa, sb):
            pltpu.async_copy(h_hbm.at[src_v.at[j, pl.ds(0, H)]],
                             buf.at[pl.ds(0, H)], sa)
            pltpu.async_copy(h_hbm.at[src_v.at[j, pl.ds(H, H)]],
                             buf.at[pl.ds(H, H)], sb)

        def wait_gather(j, buf, sa, sb):
            pltpu.make_async_copy(h_hbm.at[src_v.at[j, pl.ds(0, H)]],
                                  buf.at[pl.ds(0, H)], sa).wait()
            pltpu.make_async_copy(h_hbm.at[src_v.at[j, pl.ds(H, H)]],
                                  buf.at[pl.ds(H, H)], sb).wait()

        start_gather(0, buf0, sem0, sem0b)

        @pl.loop(0, nc, step=2)
        def _(j):
            start_gather(j + 1, buf1, sem1, sem1b)
            wait_gather(j, buf0, sem0, sem0b)
            pltpu.sync_copy(buf0, acc.at[dst_v.at[j]], add=True)

            @pl.when(j + 2 < nc)
            def _():
                start_gather(j + 2, buf0, sem0, sem0b)

            wait_gather(j + 1, buf1, sem1, sem1b)
            pltpu.sync_copy(buf1, acc.at[dst_v.at[j + 1]], add=True)

    process_section(0, SLAB)
    process_section(SLAB, CPT - SLAB)

    plsc.subcore_barrier()
    pltpu.sync_copy(acc.at[pl.ds(sid * ZPT, ZPT)],
                    out_hbm.at[cid].at[pl.ds(sid * ZPT, ZPT)])


def _tc_body(h_ref, p_ref, w_ref, b_ref, s_ref, o_ref, *, relu):
    z = s_ref[0, 0] * h_ref[...] + (p_ref[0] + p_ref[1])
    y = lax.dot_general(z, w_ref[...], (((1,), (1,)), ((), ())),
                        preferred_element_type=jnp.float32)
    y = y + b_ref[...]
    o_ref[...] = jnp.maximum(y, 0.0) if relu else y


def _tc_layer(relu):
    return pl.pallas_call(
        functools.partial(_tc_body, relu=relu),
        out_shape=jax.ShapeDtypeStruct((N, D), jnp.float32),
    )


def kernel(x, edge_index, W1, b1, eps1, W2, b2, eps2, W3, b3, eps3):
    src = edge_index[0].astype(jnp.int32)
    dst = edge_index[1].astype(jnp.int32)
    pad = E_PAD - E
    src_p = jnp.concatenate([src, jnp.zeros((pad,), jnp.int32)]).reshape(
        NW, CPT, CHUNK)
    pad_dst = N + (jnp.arange(pad, dtype=jnp.int32) % (ACC_ROWS - N))
    dst_p = jnp.concatenate([dst, pad_dst]).reshape(NW, CPT, CHUNK)
    zeros = jnp.zeros((ACC_ROWS, D), jnp.float32)

    h = x
    for W, b, eps, relu in ((W1, b1, eps1, True),
                            (W2, b2, eps2, True),
                            (W3, b3, eps3, False)):
        parts = _sc_agg(h, src_p, dst_p, zeros)[:, :N]
        scale = jnp.reshape(1.0 + eps, (1, 1)).astype(jnp.float32)
        h = _tc_layer(relu)(h, parts, W, b.reshape(1, D), scale)
    return h


# speedup vs baseline: 1.3322x; 1.3194x over previous
import functools

import jax
import jax.numpy as jnp
from jax import lax
from jax.experimental import pallas as pl
from jax.experimental.pallas import tpu as pltpu
from jax.experimental.pallas import tpu_sc as plsc

N = 10000
E = 320000
D = 128

NC = 2
NS = 16
HALF = 5056
ZROWS = 8
CHUNK = 16
SECC = 6
SECE = SECC * CHUNK
SECT = 210
EPT = SECT * SECE
E_PAD = NS * EPT
ACC_ROWS = 10112
ZPT = ACC_ROWS // NS
SPT = HALF // 8

_mesh = plsc.VectorSubcoreMesh(core_axis_name="c", subcore_axis_name="s")


@functools.partial(
    pl.kernel,
    out_type=jax.ShapeDtypeStruct((NC, ACC_ROWS, D), jnp.float32),
    mesh=_mesh,
    scratch_types=[
        pltpu.VMEM((12, CHUNK), jnp.int32),
        pltpu.VMEM((12, CHUNK), jnp.int32),
        pltpu.VMEM((CHUNK, D), jnp.float32),
        pltpu.VMEM((CHUNK, D), jnp.float32),
        pltpu.VMEM_SHARED((ACC_ROWS, D), jnp.float32),
        pltpu.VMEM_SHARED((HALF + ZROWS, D), jnp.float32),
        pltpu.SemaphoreType.DMA,
        pltpu.SemaphoreType.DMA,
        pltpu.SemaphoreType.DMA,
        pltpu.SemaphoreType.DMA,
    ],
)
def _sc_agg(h_hbm, sec_hbm, zeros_hbm, out_hbm,
            sb0, sb1, rb0, rb1, acc, staged, semA, semB, semG0, semG1):
    cid = lax.axis_index("c")
    sid = lax.axis_index("s")

    pltpu.sync_copy(zeros_hbm.at[pl.ds(sid * ZPT, ZPT)],
                    acc.at[pl.ds(sid * ZPT, ZPT)])

    @pl.when(sid < 8)
    def _():
        pltpu.sync_copy(h_hbm.at[pl.ds(cid * HALF + sid * SPT, SPT)],
                        staged.at[pl.ds(sid * SPT, SPT)])

    @pl.when(sid == 8)
    def _():
        pltpu.sync_copy(zeros_hbm.at[pl.ds(0, ZROWS)],
                        staged.at[pl.ds(HALF, ZROWS)])

    plsc.subcore_barrier()

    base = (cid * NS + sid) * SECT

    def process_section(sb):
        pltpu.async_copy(staged.at[sb.at[0]], rb0, semG0)
        for k in range(SECC):
            rb, sem = (rb0, semG0) if k % 2 == 0 else (rb1, semG1)
            nrb, nsem = (rb1, semG1) if k % 2 == 0 else (rb0, semG0)
            if k + 1 < SECC:
                pltpu.async_copy(staged.at[sb.at[k + 1]], nrb, nsem)
            pltpu.make_async_copy(staged.at[sb.at[k]], rb, sem).wait()
            pltpu.sync_copy(rb, acc.at[sb.at[6 + k]], add=True)

    pltpu.async_copy(sec_hbm.at[base], sb0, semA)
    pltpu.async_copy(sec_hbm.at[base + 1], sb1, semB)

    @pl.loop(0, SECT, step=2)
    def _(s):
        pltpu.make_async_copy(sec_hbm.at[base + s], sb0, semA).wait()
        process_section(sb0)

        @pl.when(s + 2 < SECT)
        def _():
            pltpu.async_copy(sec_hbm.at[base + s + 2], sb0, semA)

        pltpu.make_async_copy(sec_hbm.at[base + s + 1], sb1, semB).wait()
        process_section(sb1)

        @pl.when(s + 3 < SECT)
        def _():
            pltpu.async_copy(sec_hbm.at[base + s + 3], sb1, semB)

    plsc.subcore_barrier()
    pltpu.sync_copy(acc.at[pl.ds(sid * ZPT, ZPT)],
                    out_hbm.at[cid].at[pl.ds(sid * ZPT, ZPT)])


def _tc_body(h_ref, p_ref, w_ref, b_ref, s_ref, o_ref, *, relu):
    z = s_ref[0, 0] * h_ref[...] + (p_ref[0] + p_ref[1])
    y = lax.dot_general(z, w_ref[...], (((1,), (1,)), ((), ())),
                        preferred_element_type=jnp.float32)
    y = y + b_ref[...]
    o_ref[...] = jnp.maximum(y, 0.0) if relu else y


def _tc_layer(relu):
    return pl.pallas_call(
        functools.partial(_tc_body, relu=relu),
        out_shape=jax.ShapeDtypeStruct((N, D), jnp.float32),
    )


def _build_sections(src, dst):
    pad = E_PAD - E
    secs = []
    for c in range(NC):
        in_half = (src // HALF) == c
        zrow = HALF + (jnp.arange(E, dtype=jnp.int32) % ZROWS)
        gsrc = jnp.where(in_half, src - c * HALF, zrow)
        gsrc = jnp.concatenate(
            [gsrc, HALF + (jnp.arange(pad, dtype=jnp.int32) % ZROWS)])
        gdst = jnp.concatenate([dst, jnp.zeros((pad,), jnp.int32)])
        s6 = gsrc.reshape(NS, SECT, SECC, CHUNK)
        d6 = gdst.reshape(NS, SECT, SECC, CHUNK)
        secs.append(jnp.concatenate([s6, d6], axis=2))
    return jnp.stack(secs).reshape(NC * NS * SECT, 12, CHUNK)


def kernel(x, edge_index, W1, b1, eps1, W2, b2, eps2, W3, b3, eps3):
    src = edge_index[0].astype(jnp.int32)
    dst = edge_index[1].astype(jnp.int32)
    sec = _build_sections(src, dst)
    zeros = jnp.zeros((ACC_ROWS, D), jnp.float32)

    h = x
    for W, b, eps, relu in ((W1, b1, eps1, True),
                            (W2, b2, eps2, True),
                            (W3, b3, eps3, False)):
        hpad = jnp.pad(h, ((0, ACC_ROWS - N), (0, 0)))
        parts = _sc_agg(hpad, sec, zeros)[:, :N]
        scale = jnp.reshape(1.0 + eps, (1, 1)).astype(jnp.float32)
        h = _tc_layer(relu)(h, parts, W, b.reshape(1, D), scale)
    return h


# speedup vs baseline: 1.4776x; 1.1092x over previous
import functools

import jax
import jax.numpy as jnp
from jax import lax
from jax.experimental import pallas as pl
from jax.experimental.pallas import tpu as pltpu
from jax.experimental.pallas import tpu_sc as plsc

N = 10000
E = 320000
D = 128

NC = 2
NS = 16
HALF = 5056
ZROWS = 64
CHUNK = 16
SECC = 6
SECE = SECC * CHUNK
SECT = 210
EPT = SECT * SECE
E_PAD = NS * EPT
ACC_ROWS = 10112
ZPT = ACC_ROWS // NS
SPT = HALF // 8

_mesh = plsc.VectorSubcoreMesh(core_axis_name="c", subcore_axis_name="s")


@functools.partial(
    pl.kernel,
    out_type=jax.ShapeDtypeStruct((NC, ACC_ROWS, D), jnp.float32),
    mesh=_mesh,
    scratch_types=[
        pltpu.VMEM((12, CHUNK), jnp.int32),
        pltpu.VMEM((12, CHUNK), jnp.int32),
        pltpu.VMEM((CHUNK, D), jnp.float32),
        pltpu.VMEM((CHUNK, D), jnp.float32),
        pltpu.VMEM_SHARED((ACC_ROWS, D), jnp.float32),
        pltpu.VMEM_SHARED((HALF + ZROWS, D), jnp.float32),
        pltpu.SemaphoreType.DMA,
        pltpu.SemaphoreType.DMA,
        pltpu.SemaphoreType.DMA,
        pltpu.SemaphoreType.DMA,
        pltpu.SemaphoreType.DMA,
        pltpu.SemaphoreType.DMA,
    ],
)
def _sc_agg(h_hbm, sec_hbm, zeros_hbm, out_hbm,
            sb0, sb1, rb0, rb1, acc, staged,
            semA, semB, semG0, semG1, semS0, semS1):
    cid = lax.axis_index("c")
    sid = lax.axis_index("s")

    pltpu.sync_copy(zeros_hbm.at[pl.ds(sid * ZPT, ZPT)],
                    acc.at[pl.ds(sid * ZPT, ZPT)])

    @pl.when(sid < 8)
    def _():
        pltpu.sync_copy(h_hbm.at[pl.ds(cid * HALF + sid * SPT, SPT)],
                        staged.at[pl.ds(sid * SPT, SPT)])

    @pl.when(sid == 8)
    def _():
        pltpu.sync_copy(zeros_hbm.at[pl.ds(0, ZROWS)],
                        staged.at[pl.ds(HALF, ZROWS)])

    plsc.subcore_barrier()

    base = (cid * NS + sid) * SECT

    def process_section(sb, first):
        pltpu.make_async_copy(rb0, acc.at[sb.at[6]], semS0).wait()
        pltpu.async_copy(staged.at[sb.at[0]], rb0, semG0)
        for k in range(SECC):
            rb, semG, semS = ((rb0, semG0, semS0) if k % 2 == 0
                              else (rb1, semG1, semS1))
            nrb, nsemG, nsemS = ((rb1, semG1, semS1) if k % 2 == 0
                                 else (rb0, semG0, semS0))
            if k + 1 < SECC:
                pltpu.make_async_copy(nrb, acc.at[sb.at[7 + k]], nsemS).wait()
                pltpu.async_copy(staged.at[sb.at[k + 1]], nrb, nsemG)
            pltpu.make_async_copy(staged.at[sb.at[k]], rb, semG).wait()
            pltpu.async_copy(rb, acc.at[sb.at[6 + k]], semS, add=True)

    pltpu.async_copy(sec_hbm.at[base], sb0, semA)
    pltpu.async_copy(sec_hbm.at[base + 1], sb1, semB)
    pltpu.async_copy(rb0, acc.at[pl.ds(ACC_ROWS - CHUNK, CHUNK)], semS0)
    pltpu.async_copy(rb1, acc.at[pl.ds(ACC_ROWS - CHUNK, CHUNK)], semS1)

    @pl.loop(0, SECT, step=2)
    def _(s):
        pltpu.make_async_copy(sec_hbm.at[base + s], sb0, semA).wait()
        process_section(sb0, s == 0)

        @pl.when(s + 2 < SECT)
        def _():
            pltpu.async_copy(sec_hbm.at[base + s + 2], sb0, semA)

        pltpu.make_async_copy(sec_hbm.at[base + s + 1], sb1, semB).wait()
        process_section(sb1, False)

        @pl.when(s + 3 < SECT)
        def _():
            pltpu.async_copy(sec_hbm.at[base + s + 3], sb1, semB)

    pltpu.make_async_copy(rb0, acc.at[pl.ds(ACC_ROWS - CHUNK, CHUNK)],
                          semS0).wait()
    pltpu.make_async_copy(rb1, acc.at[pl.ds(ACC_ROWS - CHUNK, CHUNK)],
                          semS1).wait()
    plsc.subcore_barrier()
    pltpu.sync_copy(acc.at[pl.ds(sid * ZPT, ZPT)],
                    out_hbm.at[cid].at[pl.ds(sid * ZPT, ZPT)])


def _tc_body(h_ref, p_ref, w_ref, b_ref, s_ref, o_ref, *, relu):
    z = s_ref[0, 0] * h_ref[...] + (p_ref[0] + p_ref[1])
    y = lax.dot_general(z, w_ref[...], (((1,), (1,)), ((), ())),
                        preferred_element_type=jnp.float32)
    y = y + b_ref[...]
    o_ref[...] = jnp.maximum(y, 0.0) if relu else y


def _tc_layer(relu):
    return pl.pallas_call(
        functools.partial(_tc_body, relu=relu),
        out_shape=jax.ShapeDtypeStruct((N, D), jnp.float32),
    )


def _build_sections(src, dst):
    pad = E_PAD - E
    secs = []
    for c in range(NC):
        in_half = (src // HALF) == c
        zrow = HALF + (jnp.arange(E, dtype=jnp.int32) % ZROWS)
        gsrc = jnp.where(in_half, src - c * HALF, zrow)
        gsrc = jnp.concatenate(
            [gsrc, HALF + (jnp.arange(pad, dtype=jnp.int32) % ZROWS)])
        gdst = jnp.concatenate([dst, jnp.zeros((pad,), jnp.int32)])
        s6 = gsrc.reshape(NS, SECT, SECC, CHUNK)
        d6 = gdst.reshape(NS, SECT, SECC, CHUNK)
        secs.append(jnp.concatenate([s6, d6], axis=2))
    return jnp.stack(secs).reshape(NC * NS * SECT, 12, CHUNK)


def kernel(x, edge_index, W1, b1, eps1, W2, b2, eps2, W3, b3, eps3):
    src = edge_index[0].astype(jnp.int32)
    dst = edge_index[1].astype(jnp.int32)
    sec = _build_sections(src, dst)
    zeros = jnp.zeros((ACC_ROWS, D), jnp.float32)

    h = x
    for W, b, eps, relu in ((W1, b1, eps1, True),
                            (W2, b2, eps2, True),
                            (W3, b3, eps3, False)):
        hpad = jnp.pad(h, ((0, ACC_ROWS - N), (0, 0)))
        parts = _sc_agg(hpad, sec, zeros)[:, :N]
        scale = jnp.reshape(1.0 + eps, (1, 1)).astype(jnp.float32)
        h = _tc_layer(relu)(h, parts, W, b.reshape(1, D), scale)
    return h


# speedup vs baseline: 1.4778x; 1.0001x over previous
import functools

import jax
import jax.numpy as jnp
from jax import lax
from jax.experimental import pallas as pl
from jax.experimental.pallas import tpu as pltpu
from jax.experimental.pallas import tpu_sc as plsc

N = 10000
E = 320000
D = 128

NC = 2
NS = 16
HALF = 5056
ZROWS = 64
CHUNK = 16
SECC = 6
SECE = SECC * CHUNK
SECT = 210
EPT = SECT * SECE
E_PAD = NS * EPT
ACC_ROWS = 10112
ZPT = ACC_ROWS // NS
SPT = HALF // 8

_mesh = plsc.VectorSubcoreMesh(core_axis_name="c", subcore_axis_name="s")


@functools.partial(
    pl.kernel,
    out_type=jax.ShapeDtypeStruct((NC, ACC_ROWS, D), jnp.float32),
    mesh=_mesh,
    scratch_types=[
        pltpu.VMEM((12, CHUNK), jnp.int32),
        pltpu.VMEM((12, CHUNK), jnp.int32),
        pltpu.VMEM((CHUNK, D), jnp.float32),
        pltpu.VMEM((CHUNK, D), jnp.float32),
        pltpu.VMEM_SHARED((ACC_ROWS, D), jnp.float32),
        pltpu.VMEM_SHARED((HALF + ZROWS, D), jnp.float32),
        pltpu.SemaphoreType.DMA,
        pltpu.SemaphoreType.DMA,
        pltpu.SemaphoreType.DMA,
        pltpu.SemaphoreType.DMA,
        pltpu.SemaphoreType.DMA,
        pltpu.SemaphoreType.DMA,
    ],
)
def _sc_agg(h_hbm, sec_hbm, zeros_hbm, out_hbm,
            sb0, sb1, rb0, rb1, acc, staged,
            semA, semB, semG0, semG1, semS0, semS1):
    cid = lax.axis_index("c")
    sid = lax.axis_index("s")

    pltpu.sync_copy(zeros_hbm.at[pl.ds(sid * ZPT, ZPT)],
                    acc.at[pl.ds(sid * ZPT, ZPT)])

    @pl.when(sid < 8)
    def _():
        pltpu.sync_copy(h_hbm.at[pl.ds(cid * HALF + sid * SPT, SPT)],
                        staged.at[pl.ds(sid * SPT, SPT)])

    @pl.when(sid == 8)
    def _():
        pltpu.sync_copy(zeros_hbm.at[pl.ds(0, ZROWS)],
                        staged.at[pl.ds(HALF, ZROWS)])

    plsc.subcore_barrier()

    base = (cid * NS + sid) * SECT

    def process_section(sb):
        pltpu.make_async_copy(rb0, acc.at[sb.at[6]], semS0).wait()
        pltpu.async_copy(staged.at[sb.at[0]], rb0, semG0)
        for k in range(SECC):
            rb, semG, semS = ((rb0, semG0, semS0) if k % 2 == 0
                              else (rb1, semG1, semS1))
            nrb, nsemG, nsemS = ((rb1, semG1, semS1) if k % 2 == 0
                                 else (rb0, semG0, semS0))
            if k + 1 < SECC:
                pltpu.make_async_copy(nrb, acc.at[sb.at[7 + k]], nsemS).wait()
                pltpu.async_copy(staged.at[sb.at[k + 1]], nrb, nsemG)
            pltpu.make_async_copy(staged.at[sb.at[k]], rb, semG).wait()
            pltpu.async_copy(rb, acc.at[sb.at[6 + k]], semS, add=True)

    pltpu.async_copy(sec_hbm.at[base], sb0, semA)
    pltpu.async_copy(sec_hbm.at[base + 1], sb1, semB)
    pltpu.async_copy(rb0, acc.at[pl.ds(ACC_ROWS - CHUNK, CHUNK)], semS0)
    pltpu.async_copy(rb1, acc.at[pl.ds(ACC_ROWS - CHUNK, CHUNK)], semS1)

    @pl.loop(0, SECT, step=2)
    def _(s):
        pltpu.make_async_copy(sec_hbm.at[base + s], sb0, semA).wait()
        process_section(sb0)

        @pl.when(s + 2 < SECT)
        def _():
            pltpu.async_copy(sec_hbm.at[base + s + 2], sb0, semA)

        pltpu.make_async_copy(sec_hbm.at[base + s + 1], sb1, semB).wait()
        process_section(sb1)

        @pl.when(s + 3 < SECT)
        def _():
            pltpu.async_copy(sec_hbm.at[base + s + 3], sb1, semB)

    pltpu.make_async_copy(rb0, acc.at[pl.ds(ACC_ROWS - CHUNK, CHUNK)],
                          semS0).wait()
    pltpu.make_async_copy(rb1, acc.at[pl.ds(ACC_ROWS - CHUNK, CHUNK)],
                          semS1).wait()
    plsc.subcore_barrier()
    pltpu.sync_copy(acc.at[pl.ds(sid * ZPT, ZPT)],
                    out_hbm.at[cid].at[pl.ds(sid * ZPT, ZPT)])


def _tc_body(h_ref, p_ref, w_ref, b_ref, s_ref, o_ref, *, relu):
    z = s_ref[0, 0] * h_ref[...] + (p_ref[0] + p_ref[1])
    y = lax.dot_general(z, w_ref[...], (((1,), (1,)), ((), ())),
                        preferred_element_type=jnp.float32)
    y = y + b_ref[...]
    o_ref[...] = jnp.maximum(y, 0.0) if relu else y


def _tc_layer(relu):
    return pl.pallas_call(
        functools.partial(_tc_body, relu=relu),
        out_shape=jax.ShapeDtypeStruct((N, D), jnp.float32),
    )


def _build_sections(src, dst):
    pad = E_PAD - E
    secs = []
    for c in range(NC):
        in_half = (src // HALF) == c
        zrow = HALF + (jnp.arange(E, dtype=jnp.int32) % ZROWS)
        gsrc = jnp.where(in_half, src - c * HALF, zrow)
        gsrc = jnp.concatenate(
            [gsrc, HALF + (jnp.arange(pad, dtype=jnp.int32) % ZROWS)])
        gdst = jnp.concatenate([dst, jnp.zeros((pad,), jnp.int32)])
        s6 = gsrc.reshape(NS, SECT, SECC, CHUNK)
        d6 = gdst.reshape(NS, SECT, SECC, CHUNK)
        secs.append(jnp.concatenate([s6, d6], axis=2))
    return jnp.stack(secs).reshape(NC * NS * SECT, 12, CHUNK)


def kernel(x, edge_index, W1, b1, eps1, W2, b2, eps2, W3, b3, eps3):
    src = edge_index[0].astype(jnp.int32)
    dst = edge_index[1].astype(jnp.int32)
    sec = _build_sections(src, dst)
    zeros = jnp.zeros((ACC_ROWS, D), jnp.float32)

    h = x
    for W, b, eps, relu in ((W1, b1, eps1, True),
                            (W2, b2, eps2, True),
                            (W3, b3, eps3, False)):
        hpad = jnp.pad(h, ((0, ACC_ROWS - N), (0, 0)))
        parts = _sc_agg(hpad, sec, zeros)[:, :N]
        scale = jnp.reshape(1.0 + eps, (1, 1)).astype(jnp.float32)
        h = _tc_layer(relu)(h, parts, W, b.reshape(1, D), scale)
    return h
